# baseline (speedup 1.0000x reference)
import jax
import jax.numpy as jnp
from jax import lax
from jax.experimental import pallas as pl
from jax.experimental.pallas import tpu as pltpu
from jax.experimental.pallas import tpu_sc as plsc

N_NODES = 10000
N_EDGES = 320000
D_NODE = 128
D_EDGE = 16
D_GLOBAL = 64
HIDDEN = 128

NUM_CORES = 2
NUM_SUBCORES = 16
NUM_WORKERS = NUM_CORES * NUM_SUBCORES

EDGES_PER_WORKER = N_EDGES // NUM_WORKERS
N_CHUNKS = 5
EDGES_PER_CALL = N_EDGES // N_CHUNKS
EDGES_PER_WORKER_CALL = EDGES_PER_CALL // NUM_WORKERS
GATHER_CHUNK = 80
FULL_CHUNKS = EDGES_PER_WORKER_CALL // GATHER_CHUNK


def _precompute_body(nodes_ref, w1r_ref, w1s_ref, w1g_ref, glbls_ref, b1_ref,
                     p_ref, q_ref):
    c = jnp.dot(glbls_ref[...], w1g_ref[...],
                preferred_element_type=jnp.float32) + b1_ref[...]
    nodes = nodes_ref[...]
    p_ref[...] = jnp.dot(nodes, w1r_ref[...],
                         preferred_element_type=jnp.float32) + c
    q_ref[...] = jnp.dot(nodes, w1s_ref[...],
                         preferred_element_type=jnp.float32)


def _precompute(nodes, w1r, w1s, w1g, glbls, b1):
    return pl.pallas_call(
        _precompute_body,
        out_shape=(
            jax.ShapeDtypeStruct((N_NODES, HIDDEN), jnp.float32),
            jax.ShapeDtypeStruct((N_NODES, HIDDEN), jnp.float32),
        ),
    )(nodes, w1r, w1s, w1g, glbls, b1)


def _sc_gather_body(p_hbm, q_hbm, recv_hbm, send_hbm, gr_hbm, gs_hbm,
                    recv_v, send_v, rows_pa, rows_qa, rows_pb, rows_qb,
                    sem_pa, sem_qa, sem_pb, sem_qb, sem_wpa, sem_wqa,
                    sem_wpb, sem_wqb):
    wid = lax.axis_index("s") * NUM_CORES + lax.axis_index("c")
    base = wid * EDGES_PER_WORKER_CALL

    pltpu.sync_copy(recv_hbm.at[pl.ds(base, EDGES_PER_WORKER_CALL)], recv_v)
    pltpu.sync_copy(send_hbm.at[pl.ds(base, EDGES_PER_WORKER_CALL)], send_v)

    def gather(j, rows_p, rows_q, sem_p, sem_q):
        off = j * GATHER_CHUNK
        pltpu.async_copy(
            p_hbm.at[recv_v.at[pl.ds(off, GATHER_CHUNK)]], rows_p, sem_p)
        pltpu.async_copy(
            q_hbm.at[send_v.at[pl.ds(off, GATHER_CHUNK)]], rows_q, sem_q)

    def wait_gather(j, rows_p, rows_q, sem_p, sem_q):
        off = j * GATHER_CHUNK
        pltpu.make_async_copy(
            p_hbm.at[recv_v.at[pl.ds(off, GATHER_CHUNK)]], rows_p,
            sem_p).wait()
        pltpu.make_async_copy(
            q_hbm.at[send_v.at[pl.ds(off, GATHER_CHUNK)]], rows_q,
            sem_q).wait()

    def store(j, rows_p, rows_q, sem_wp, sem_wq):
        off = base + j * GATHER_CHUNK
        pltpu.async_copy(rows_p, gr_hbm.at[pl.ds(off, GATHER_CHUNK)], sem_wp)
        pltpu.async_copy(rows_q, gs_hbm.at[pl.ds(off, GATHER_CHUNK)], sem_wq)

    def wait_store(j, rows_p, rows_q, sem_wp, sem_wq):
        off = base + j * GATHER_CHUNK
        pltpu.make_async_copy(
            rows_p, gr_hbm.at[pl.ds(off, GATHER_CHUNK)], sem_wp).wait()
        pltpu.make_async_copy(
            rows_q, gs_hbm.at[pl.ds(off, GATHER_CHUNK)], sem_wq).wait()

    gather(0, rows_pa, rows_qa, sem_pa, sem_qa)
    gather(1, rows_pb, rows_qb, sem_pb, sem_qb)

    def body(i, carry):
        j = 2 * i
        wait_gather(j, rows_pa, rows_qa, sem_pa, sem_qa)
        store(j, rows_pa, rows_qa, sem_wpa, sem_wqa)
        wait_gather(j + 1, rows_pb, rows_qb, sem_pb, sem_qb)
        store(j + 1, rows_pb, rows_qb, sem_wpb, sem_wqb)
        wait_store(j, rows_pa, rows_qa, sem_wpa, sem_wqa)
        gather(j + 2, rows_pa, rows_qa, sem_pa, sem_qa)
        wait_store(j + 1, rows_pb, rows_qb, sem_wpb, sem_wqb)
        gather(j + 3, rows_pb, rows_qb, sem_pb, sem_qb)
        return carry

    lax.fori_loop(0, (FULL_CHUNKS - 3) // 2, body, 0, unroll=False)
    j = FULL_CHUNKS - 3
    wait_gather(j, rows_pa, rows_qa, sem_pa, sem_qa)
    store(j, rows_pa, rows_qa, sem_wpa, sem_wqa)
    wait_gather(j + 1, rows_pb, rows_qb, sem_pb, sem_qb)
    store(j + 1, rows_pb, rows_qb, sem_wpb, sem_wqb)
    wait_store(j, rows_pa, rows_qa, sem_wpa, sem_wqa)
    gather(j + 2, rows_pa, rows_qa, sem_pa, sem_qa)
    wait_gather(j + 2, rows_pa, rows_qa, sem_pa, sem_qa)
    store(j + 2, rows_pa, rows_qa, sem_wpa, sem_wqa)
    wait_store(j + 1, rows_pb, rows_qb, sem_wpb, sem_wqb)
    wait_store(j + 2, rows_pa, rows_qa, sem_wpa, sem_wqa)


def _sc_gather(p, q, receivers, senders):
    mesh = plsc.VectorSubcoreMesh(core_axis_name="c", subcore_axis_name="s",
                                  num_cores=NUM_CORES,
                                  num_subcores=NUM_SUBCORES)
    return pl.kernel(
        _sc_gather_body,
        out_type=(
            jax.ShapeDtypeStruct((EDGES_PER_CALL, HIDDEN), jnp.float32),
            jax.ShapeDtypeStruct((EDGES_PER_CALL, HIDDEN), jnp.float32),
        ),
        mesh=mesh,
        scratch_types=[
            pltpu.VMEM((EDGES_PER_WORKER_CALL,), jnp.int32),
            pltpu.VMEM((EDGES_PER_WORKER_CALL,), jnp.int32),
            pltpu.VMEM((GATHER_CHUNK, HIDDEN), jnp.float32),
            pltpu.VMEM((GATHER_CHUNK, HIDDEN), jnp.float32),
            pltpu.VMEM((GATHER_CHUNK, HIDDEN), jnp.float32),
            pltpu.VMEM((GATHER_CHUNK, HIDDEN), jnp.float32),
            pltpu.SemaphoreType.DMA,
            pltpu.SemaphoreType.DMA,
            pltpu.SemaphoreType.DMA,
            pltpu.SemaphoreType.DMA,
            pltpu.SemaphoreType.DMA,
            pltpu.SemaphoreType.DMA,
            pltpu.SemaphoreType.DMA,
            pltpu.SemaphoreType.DMA,
        ],
    )(p, q, receivers, senders)


EDGE_BLOCK = 4000


def _mlp_body(gr_ref, gs_ref, edges_ref, w1e_ref, w2_ref, b2_ref, out_ref):
    h = gr_ref[...] + gs_ref[...] + jnp.dot(
        edges_ref[...], w1e_ref[...], preferred_element_type=jnp.float32)
    h = jnp.maximum(h, 0.0)
    out_ref[...] = jnp.dot(h, w2_ref[...],
                           preferred_element_type=jnp.float32) + b2_ref[...]


def _edge_mlp(gr, gs, edges, w1e, w2, b2, chunk):
    grid = (EDGES_PER_CALL // EDGE_BLOCK,)
    blk0 = chunk * (EDGES_PER_CALL // EDGE_BLOCK)
    return pl.pallas_call(
        _mlp_body,
        grid=grid,
        in_specs=[
            pl.BlockSpec((EDGE_BLOCK, HIDDEN), lambda i: (i, 0)),
            pl.BlockSpec((EDGE_BLOCK, HIDDEN), lambda i: (i, 0)),
            pl.BlockSpec((EDGE_BLOCK, D_EDGE), lambda i: (blk0 + i, 0)),
            pl.BlockSpec((D_EDGE, HIDDEN), lambda i: (0, 0)),
            pl.BlockSpec((HIDDEN, D_EDGE), lambda i: (0, 0)),
            pl.BlockSpec((1, D_EDGE), lambda i: (0, 0)),
        ],
        out_specs=pl.BlockSpec((EDGE_BLOCK, D_EDGE), lambda i: (i, 0)),
        out_shape=jax.ShapeDtypeStruct((EDGES_PER_CALL, D_EDGE), jnp.float32),
        compiler_params=pltpu.CompilerParams(
            dimension_semantics=("arbitrary",)),
    )(gr, gs, edges, w1e, w2, b2)


@jax.jit
def kernel(edges, nodes, glbls, W1, b1, W2, b2, senders, receivers):
    w1e = W1[:D_EDGE]
    w1r = W1[D_EDGE:D_EDGE + D_NODE]
    w1s = W1[D_EDGE + D_NODE:D_EDGE + 2 * D_NODE]
    w1g = W1[D_EDGE + 2 * D_NODE:]
    p, q = _precompute(nodes, w1r, w1s, w1g, glbls, b1.reshape(1, HIDDEN))
    b2r = b2.reshape(1, D_EDGE)
    out = jnp.zeros((N_EDGES, D_EDGE), jnp.float32)
    for c in range(N_CHUNKS):
        lo = c * EDGES_PER_CALL
        rc = lax.slice_in_dim(receivers, lo, lo + EDGES_PER_CALL)
        sc = lax.slice_in_dim(senders, lo, lo + EDGES_PER_CALL)
        gr, gs = _sc_gather(p, q, rc, sc)
        out_c = _edge_mlp(gr, gs, edges, w1e, W2, b2r, c)
        out = lax.dynamic_update_slice(out, out_c, (lo, 0))
    return out


# speedup vs baseline: 1.3534x; 1.3534x over previous
import jax
import jax.numpy as jnp
from jax import lax
from jax.experimental import pallas as pl
from jax.experimental.pallas import tpu as pltpu
from jax.experimental.pallas import tpu_sc as plsc

N_NODES = 10000
N_EDGES = 320000
D_NODE = 128
D_EDGE = 16
D_GLOBAL = 64
HIDDEN = 128
LANES = 16

NUM_CORES = 2
NUM_SUBCORES = 16
NUM_WORKERS = NUM_CORES * NUM_SUBCORES

N_CHUNKS = 2
EDGES_PER_CALL = N_EDGES // N_CHUNKS
EDGES_PER_WORKER_CALL = EDGES_PER_CALL // NUM_WORKERS
GATHER_CHUNK = 40
FULL_CHUNKS = EDGES_PER_WORKER_CALL // GATHER_CHUNK


def _precompute_body(nodes_ref, w1r_ref, w1s_ref, w1g_ref, glbls_ref, b1_ref,
                     p_ref, q_ref):
    c = jnp.dot(glbls_ref[...], w1g_ref[...],
                preferred_element_type=jnp.float32) + b1_ref[...]
    nodes = nodes_ref[...]
    p_ref[...] = jnp.dot(nodes, w1r_ref[...],
                         preferred_element_type=jnp.float32) + c
    q_ref[...] = jnp.dot(nodes, w1s_ref[...],
                         preferred_element_type=jnp.float32)


def _precompute(nodes, w1r, w1s, w1g, glbls, b1):
    return pl.pallas_call(
        _precompute_body,
        out_shape=(
            jax.ShapeDtypeStruct((N_NODES, HIDDEN), jnp.float32),
            jax.ShapeDtypeStruct((N_NODES, HIDDEN), jnp.float32),
        ),
    )(nodes, w1r, w1s, w1g, glbls, b1)


def _sc_gather_body(p_hbm, q_hbm, recv_hbm, send_hbm, g_hbm,
                    recv_v, send_v, rows_pa, rows_qa, rows_pb, rows_qb,
                    sem_pa, sem_qa, sem_pb, sem_qb, sem_wa, sem_wb):
    wid = lax.axis_index("s") * NUM_CORES + lax.axis_index("c")
    base = wid * EDGES_PER_WORKER_CALL

    pltpu.sync_copy(recv_hbm.at[pl.ds(base, EDGES_PER_WORKER_CALL)], recv_v)
    pltpu.sync_copy(send_hbm.at[pl.ds(base, EDGES_PER_WORKER_CALL)], send_v)

    def gather(j, rows_p, rows_q, sem_p, sem_q):
        off = j * GATHER_CHUNK
        pltpu.async_copy(
            p_hbm.at[recv_v.at[pl.ds(off, GATHER_CHUNK)]], rows_p, sem_p)
        pltpu.async_copy(
            q_hbm.at[send_v.at[pl.ds(off, GATHER_CHUNK)]], rows_q, sem_q)

    def wait_gather(j, rows_p, rows_q, sem_p, sem_q):
        off = j * GATHER_CHUNK
        pltpu.make_async_copy(
            p_hbm.at[recv_v.at[pl.ds(off, GATHER_CHUNK)]], rows_p,
            sem_p).wait()
        pltpu.make_async_copy(
            q_hbm.at[send_v.at[pl.ds(off, GATHER_CHUNK)]], rows_q,
            sem_q).wait()

    def add_rows(rows_p, rows_q):
        def row(r, carry):
            for cc in range(HIDDEN // LANES):
                sl = pl.ds(cc * LANES, LANES)
                plsc.addupdate(rows_p.at[r, sl], rows_q[r, sl])
            return carry
        lax.fori_loop(0, GATHER_CHUNK, row, 0, unroll=4)

    def store(j, rows_p, sem_w):
        off = base + j * GATHER_CHUNK
        pltpu.async_copy(rows_p, g_hbm.at[pl.ds(off, GATHER_CHUNK)], sem_w)

    def wait_store(j, rows_p, sem_w):
        off = base + j * GATHER_CHUNK
        pltpu.make_async_copy(
            rows_p, g_hbm.at[pl.ds(off, GATHER_CHUNK)], sem_w).wait()

    gather(0, rows_pa, rows_qa, sem_pa, sem_qa)
    gather(1, rows_pb, rows_qb, sem_pb, sem_qb)

    def body(i, carry):
        j = 2 * i
        wait_gather(j, rows_pa, rows_qa, sem_pa, sem_qa)
        add_rows(rows_pa, rows_qa)
        store(j, rows_pa, sem_wa)
        wait_gather(j + 1, rows_pb, rows_qb, sem_pb, sem_qb)
        add_rows(rows_pb, rows_qb)
        store(j + 1, rows_pb, sem_wb)
        wait_store(j, rows_pa, sem_wa)
        gather(j + 2, rows_pa, rows_qa, sem_pa, sem_qa)
        wait_store(j + 1, rows_pb, sem_wb)
        gather(j + 3, rows_pb, rows_qb, sem_pb, sem_qb)
        return carry

    lax.fori_loop(0, (FULL_CHUNKS - 3) // 2, body, 0, unroll=False)
    j = FULL_CHUNKS - 3
    wait_gather(j, rows_pa, rows_qa, sem_pa, sem_qa)
    add_rows(rows_pa, rows_qa)
    store(j, rows_pa, sem_wa)
    wait_gather(j + 1, rows_pb, rows_qb, sem_pb, sem_qb)
    add_rows(rows_pb, rows_qb)
    store(j + 1, rows_pb, sem_wb)
    wait_store(j, rows_pa, sem_wa)
    gather(j + 2, rows_pa, rows_qa, sem_pa, sem_qa)
    wait_gather(j + 2, rows_pa, rows_qa, sem_pa, sem_qa)
    add_rows(rows_pa, rows_qa)
    store(j + 2, rows_pa, sem_wa)
    wait_store(j + 1, rows_pb, sem_wb)
    wait_store(j + 2, rows_pa, sem_wa)


def _sc_gather(p, q, receivers, senders):
    mesh = plsc.VectorSubcoreMesh(core_axis_name="c", subcore_axis_name="s",
                                  num_cores=NUM_CORES,
                                  num_subcores=NUM_SUBCORES)
    return pl.kernel(
        _sc_gather_body,
        out_type=jax.ShapeDtypeStruct((EDGES_PER_CALL, HIDDEN), jnp.float32),
        mesh=mesh,
        scratch_types=[
            pltpu.VMEM((EDGES_PER_WORKER_CALL,), jnp.int32),
            pltpu.VMEM((EDGES_PER_WORKER_CALL,), jnp.int32),
            pltpu.VMEM((GATHER_CHUNK, HIDDEN), jnp.float32),
            pltpu.VMEM((GATHER_CHUNK, HIDDEN), jnp.float32),
            pltpu.VMEM((GATHER_CHUNK, HIDDEN), jnp.float32),
            pltpu.VMEM((GATHER_CHUNK, HIDDEN), jnp.float32),
            pltpu.SemaphoreType.DMA,
            pltpu.SemaphoreType.DMA,
            pltpu.SemaphoreType.DMA,
            pltpu.SemaphoreType.DMA,
            pltpu.SemaphoreType.DMA,
            pltpu.SemaphoreType.DMA,
        ],
    )(p, q, receivers, senders)


EDGE_BLOCK = 4000


def _mlp_first_body(g_ref, edges_ref, w1e_ref, w2_ref, b2_ref, out_ref):
    h = g_ref[...] + jnp.dot(
        edges_ref[...], w1e_ref[...], preferred_element_type=jnp.float32)
    h = jnp.maximum(h, 0.0)
    out_ref[...] = jnp.dot(h, w2_ref[...],
                           preferred_element_type=jnp.float32) + b2_ref[...]


def _mlp_chain_body(g_ref, edges_ref, w1e_ref, w2_ref, b2_ref, prev_ref,
                    out_ref):
    h = g_ref[...] + jnp.dot(
        edges_ref[...], w1e_ref[...], preferred_element_type=jnp.float32)
    h = jnp.maximum(h, 0.0)
    out_ref[...] = jnp.dot(h, w2_ref[...],
                           preferred_element_type=jnp.float32) + b2_ref[...]


def _edge_mlp(g, edges, w1e, w2, b2, chunk, prev):
    grid = (EDGES_PER_CALL // EDGE_BLOCK,)
    blk0 = chunk * (EDGES_PER_CALL // EDGE_BLOCK)
    common = dict(
        grid=grid,
        out_specs=pl.BlockSpec((EDGE_BLOCK, D_EDGE), lambda i: (blk0 + i, 0)),
        out_shape=jax.ShapeDtypeStruct((N_EDGES, D_EDGE), jnp.float32),
        compiler_params=pltpu.CompilerParams(
            dimension_semantics=("arbitrary",)),
    )
    in_specs = [
        pl.BlockSpec((EDGE_BLOCK, HIDDEN), lambda i: (i, 0)),
        pl.BlockSpec((EDGE_BLOCK, D_EDGE), lambda i: (blk0 + i, 0)),
        pl.BlockSpec((D_EDGE, HIDDEN), lambda i: (0, 0)),
        pl.BlockSpec((HIDDEN, D_EDGE), lambda i: (0, 0)),
        pl.BlockSpec((1, D_EDGE), lambda i: (0, 0)),
    ]
    if prev is None:
        return pl.pallas_call(
            _mlp_first_body, in_specs=in_specs, **common,
        )(g, edges, w1e, w2, b2)
    in_specs.append(pl.BlockSpec((8, D_EDGE), lambda i: (0, 0)))
    return pl.pallas_call(
        _mlp_chain_body, in_specs=in_specs, input_output_aliases={5: 0},
        **common,
    )(g, edges, w1e, w2, b2, prev)


@jax.jit
def kernel(edges, nodes, glbls, W1, b1, W2, b2, senders, receivers):
    w1e = W1[:D_EDGE]
    w1r = W1[D_EDGE:D_EDGE + D_NODE]
    w1s = W1[D_EDGE + D_NODE:D_EDGE + 2 * D_NODE]
    w1g = W1[D_EDGE + 2 * D_NODE:]
    p, q = _precompute(nodes, w1r, w1s, w1g, glbls, b1.reshape(1, HIDDEN))
    b2r = b2.reshape(1, D_EDGE)

    out = None
    for c in range(N_CHUNKS):
        lo = c * EDGES_PER_CALL
        rc = lax.slice_in_dim(receivers, lo, lo + EDGES_PER_CALL)
        sc = lax.slice_in_dim(senders, lo, lo + EDGES_PER_CALL)
        g = _sc_gather(p, q, rc, sc)
        out = _edge_mlp(g, edges, w1e, W2, b2r, c, out)
    return out


# speedup vs baseline: 1.9408x; 1.4340x over previous
import jax
import jax.numpy as jnp
from jax import lax
from jax.experimental import pallas as pl
from jax.experimental.pallas import tpu as pltpu
from jax.experimental.pallas import tpu_sc as plsc

N_NODES = 10000
N_EDGES = 320000
D_NODE = 128
D_EDGE = 16
D_GLOBAL = 64
HIDDEN = 128
LANES = 16

NUM_CORES = 2
NUM_SUBCORES = 16
NUM_WORKERS = NUM_CORES * NUM_SUBCORES

N_CHUNKS = 2
EDGES_PER_CALL = N_EDGES // N_CHUNKS
EDGES_PER_WORKER_CALL = EDGES_PER_CALL // NUM_WORKERS
GATHER_CHUNK = 40
FULL_CHUNKS = EDGES_PER_WORKER_CALL // GATHER_CHUNK


def _precompute_body(nodes_ref, w1r_ref, w1s_ref, w1g_ref, glbls_ref, b1_ref,
                     p_ref, q_ref):
    c = jnp.dot(glbls_ref[...], w1g_ref[...],
                preferred_element_type=jnp.float32) + b1_ref[...]
    nodes = nodes_ref[...]
    p_ref[...] = jnp.dot(nodes, w1r_ref[...],
                         preferred_element_type=jnp.float32) + c
    q_ref[...] = jnp.dot(nodes, w1s_ref[...],
                         preferred_element_type=jnp.float32)


def _precompute(nodes, w1r, w1s, w1g, glbls, b1):
    return pl.pallas_call(
        _precompute_body,
        out_shape=(
            jax.ShapeDtypeStruct((N_NODES, HIDDEN), jnp.float32),
            jax.ShapeDtypeStruct((N_NODES, HIDDEN), jnp.float32),
        ),
    )(nodes, w1r, w1s, w1g, glbls, b1)


def _sc_gather_body(p_hbm, q_hbm, recv_hbm, send_hbm, g_hbm,
                    recv_v, send_v, rows_pa, rows_qa, rows_pb, rows_qb,
                    sem_pa, sem_qa, sem_pb, sem_qb, sem_wa, sem_wb):
    wid = lax.axis_index("s") * NUM_CORES + lax.axis_index("c")
    base = wid * EDGES_PER_WORKER_CALL

    pltpu.sync_copy(recv_hbm.at[pl.ds(base, EDGES_PER_WORKER_CALL)], recv_v)
    pltpu.sync_copy(send_hbm.at[pl.ds(base, EDGES_PER_WORKER_CALL)], send_v)

    def gather(j, rows_p, rows_q, sem_p, sem_q):
        off = j * GATHER_CHUNK
        pltpu.async_copy(
            p_hbm.at[recv_v.at[pl.ds(off, GATHER_CHUNK)]], rows_p, sem_p)
        pltpu.async_copy(
            q_hbm.at[send_v.at[pl.ds(off, GATHER_CHUNK)]], rows_q, sem_q)

    def wait_gather(j, rows_p, rows_q, sem_p, sem_q):
        off = j * GATHER_CHUNK
        pltpu.make_async_copy(
            p_hbm.at[recv_v.at[pl.ds(off, GATHER_CHUNK)]], rows_p,
            sem_p).wait()
        pltpu.make_async_copy(
            q_hbm.at[send_v.at[pl.ds(off, GATHER_CHUNK)]], rows_q,
            sem_q).wait()

    def add_rows(rows_p, rows_q):
        def row(r, carry):
            for cc in range(HIDDEN // LANES):
                sl = pl.ds(cc * LANES, LANES)
                plsc.addupdate(rows_p.at[r, sl], rows_q[r, sl])
            return carry
        lax.fori_loop(0, GATHER_CHUNK, row, 0, unroll=4)

    def store(j, rows_p, sem_w):
        off = base + j * GATHER_CHUNK
        pltpu.async_copy(rows_p, g_hbm.at[pl.ds(off, GATHER_CHUNK)], sem_w)

    def wait_store(j, rows_p, sem_w):
        off = base + j * GATHER_CHUNK
        pltpu.make_async_copy(
            rows_p, g_hbm.at[pl.ds(off, GATHER_CHUNK)], sem_w).wait()

    gather(0, rows_pa, rows_qa, sem_pa, sem_qa)
    gather(1, rows_pb, rows_qb, sem_pb, sem_qb)

    def body(i, carry):
        j = 2 * i
        wait_gather(j, rows_pa, rows_qa, sem_pa, sem_qa)
        add_rows(rows_pa, rows_qa)
        store(j, rows_pa, sem_wa)
        wait_gather(j + 1, rows_pb, rows_qb, sem_pb, sem_qb)
        add_rows(rows_pb, rows_qb)
        store(j + 1, rows_pb, sem_wb)
        wait_store(j, rows_pa, sem_wa)
        gather(j + 2, rows_pa, rows_qa, sem_pa, sem_qa)
        wait_store(j + 1, rows_pb, sem_wb)
        gather(j + 3, rows_pb, rows_qb, sem_pb, sem_qb)
        return carry

    lax.fori_loop(0, (FULL_CHUNKS - 3) // 2, body, 0, unroll=False)
    j = FULL_CHUNKS - 3
    wait_gather(j, rows_pa, rows_qa, sem_pa, sem_qa)
    add_rows(rows_pa, rows_qa)
    store(j, rows_pa, sem_wa)
    wait_gather(j + 1, rows_pb, rows_qb, sem_pb, sem_qb)
    add_rows(rows_pb, rows_qb)
    store(j + 1, rows_pb, sem_wb)
    wait_store(j, rows_pa, sem_wa)
    gather(j + 2, rows_pa, rows_qa, sem_pa, sem_qa)
    wait_gather(j + 2, rows_pa, rows_qa, sem_pa, sem_qa)
    add_rows(rows_pa, rows_qa)
    store(j + 2, rows_pa, sem_wa)
    wait_store(j + 1, rows_pb, sem_wb)
    wait_store(j + 2, rows_pa, sem_wa)


def _sc_gather(p, q, receivers, senders):
    mesh = plsc.VectorSubcoreMesh(core_axis_name="c", subcore_axis_name="s",
                                  num_cores=NUM_CORES,
                                  num_subcores=NUM_SUBCORES)
    return pl.kernel(
        _sc_gather_body,
        out_type=jax.ShapeDtypeStruct((EDGES_PER_CALL, HIDDEN), jnp.float32),
        mesh=mesh,
        scratch_types=[
            pltpu.VMEM((EDGES_PER_WORKER_CALL,), jnp.int32),
            pltpu.VMEM((EDGES_PER_WORKER_CALL,), jnp.int32),
            pltpu.VMEM((GATHER_CHUNK, HIDDEN), jnp.float32),
            pltpu.VMEM((GATHER_CHUNK, HIDDEN), jnp.float32),
            pltpu.VMEM((GATHER_CHUNK, HIDDEN), jnp.float32),
            pltpu.VMEM((GATHER_CHUNK, HIDDEN), jnp.float32),
            pltpu.SemaphoreType.DMA,
            pltpu.SemaphoreType.DMA,
            pltpu.SemaphoreType.DMA,
            pltpu.SemaphoreType.DMA,
            pltpu.SemaphoreType.DMA,
            pltpu.SemaphoreType.DMA,
        ],
    )(p, q, receivers, senders)


EDGE_BLOCK = 3200


def _mlp_compute(g_ref, edges_t_ref, w1e_ref, w2_ref, b2_ref, out_ref):
    t = lax.dot_general(edges_t_ref[...], w1e_ref[...],
                        dimension_numbers=((([0]), ([0])), ((), ())),
                        preferred_element_type=jnp.float32)
    h = jnp.maximum(g_ref[...] + t, 0.0)
    out_t = lax.dot_general(w2_ref[...], h,
                            dimension_numbers=((([0]), ([1])), ((), ())),
                            preferred_element_type=jnp.float32)
    out_ref[...] = out_t + b2_ref[:, 0:1]


def _mlp_first_body(g_ref, edges_t_ref, w1e_ref, w2_ref, b2_ref, out_ref):
    _mlp_compute(g_ref, edges_t_ref, w1e_ref, w2_ref, b2_ref, out_ref)


def _mlp_chain_body(g_ref, edges_t_ref, w1e_ref, w2_ref, b2_ref, prev_ref,
                    out_ref):
    _mlp_compute(g_ref, edges_t_ref, w1e_ref, w2_ref, b2_ref, out_ref)


def _edge_mlp(g, edges_t, w1e, w2, b2col, chunk, prev):
    grid = (EDGES_PER_CALL // EDGE_BLOCK,)
    blk0 = chunk * (EDGES_PER_CALL // EDGE_BLOCK)
    common = dict(
        grid=grid,
        out_specs=pl.BlockSpec((D_EDGE, EDGE_BLOCK), lambda i: (0, blk0 + i)),
        out_shape=jax.ShapeDtypeStruct((D_EDGE, N_EDGES), jnp.float32),
        compiler_params=pltpu.CompilerParams(
            dimension_semantics=("arbitrary",)),
    )
    in_specs = [
        pl.BlockSpec((EDGE_BLOCK, HIDDEN), lambda i: (i, 0)),
        pl.BlockSpec((D_EDGE, EDGE_BLOCK), lambda i: (0, blk0 + i)),
        pl.BlockSpec((D_EDGE, HIDDEN), lambda i: (0, 0)),
        pl.BlockSpec((HIDDEN, D_EDGE), lambda i: (0, 0)),
        pl.BlockSpec((D_EDGE, 128), lambda i: (0, 0)),
    ]
    if prev is None:
        return pl.pallas_call(
            _mlp_first_body, in_specs=in_specs, **common,
        )(g, edges_t, w1e, w2, b2col)
    in_specs.append(pl.BlockSpec((D_EDGE, 128), lambda i: (0, 0)))
    return pl.pallas_call(
        _mlp_chain_body, in_specs=in_specs, input_output_aliases={5: 0},
        **common,
    )(g, edges_t, w1e, w2, b2col, prev)


@jax.jit
def kernel(edges, nodes, glbls, W1, b1, W2, b2, senders, receivers):
    w1e = W1[:D_EDGE]
    w1r = W1[D_EDGE:D_EDGE + D_NODE]
    w1s = W1[D_EDGE + D_NODE:D_EDGE + 2 * D_NODE]
    w1g = W1[D_EDGE + 2 * D_NODE:]
    p, q = _precompute(nodes, w1r, w1s, w1g, glbls, b1.reshape(1, HIDDEN))
    b2col = jnp.broadcast_to(b2.reshape(D_EDGE, 1), (D_EDGE, 128))
    edges_t = edges.T

    out_t = None
    for c in range(N_CHUNKS):
        lo = c * EDGES_PER_CALL
        rc = lax.slice_in_dim(receivers, lo, lo + EDGES_PER_CALL)
        sc = lax.slice_in_dim(senders, lo, lo + EDGES_PER_CALL)
        g = _sc_gather(p, q, rc, sc)
        out_t = _edge_mlp(g, edges_t, w1e, W2, b2col, c, out_t)
    return out_t.T


# speedup vs baseline: 2.0561x; 1.0594x over previous
import jax
import jax.numpy as jnp
from jax import lax
from jax.experimental import pallas as pl
from jax.experimental.pallas import tpu as pltpu
from jax.experimental.pallas import tpu_sc as plsc

N_NODES = 10000
N_EDGES = 320000
D_NODE = 128
D_EDGE = 16
D_GLOBAL = 64
HIDDEN = 128
LANES = 16

NUM_CORES = 2
NUM_SUBCORES = 16
NUM_WORKERS = NUM_CORES * NUM_SUBCORES

SLICE_SIZES = (160000, 96000, 64000)
GATHER_CHUNK = 40


def _precompute_body(nodes_ref, w1r_ref, w1s_ref, w1g_ref, glbls_ref, b1_ref,
                     p_ref, q_ref):
    c = jnp.dot(glbls_ref[...], w1g_ref[...],
                preferred_element_type=jnp.float32) + b1_ref[...]
    nodes = nodes_ref[...]
    p_ref[...] = jnp.dot(nodes, w1r_ref[...],
                         preferred_element_type=jnp.float32) + c
    q_ref[...] = jnp.dot(nodes, w1s_ref[...],
                         preferred_element_type=jnp.float32)


def _precompute(nodes, w1r, w1s, w1g, glbls, b1):
    return pl.pallas_call(
        _precompute_body,
        out_shape=(
            jax.ShapeDtypeStruct((N_NODES, HIDDEN), jnp.float32),
            jax.ShapeDtypeStruct((N_NODES, HIDDEN), jnp.float32),
        ),
    )(nodes, w1r, w1s, w1g, glbls, b1)


def _make_sc_body(worker_edges):
    full_chunks = worker_edges // GATHER_CHUNK

    def sc_body(p_hbm, q_hbm, recv_hbm, send_hbm, g_hbm,
                recv_v, send_v, rows_pa, rows_qa, rows_pb, rows_qb,
                sem_pa, sem_qa, sem_pb, sem_qb, sem_wa, sem_wb):
        wid = lax.axis_index("s") * NUM_CORES + lax.axis_index("c")
        base = wid * worker_edges

        pltpu.sync_copy(recv_hbm.at[pl.ds(base, worker_edges)], recv_v)
        pltpu.sync_copy(send_hbm.at[pl.ds(base, worker_edges)], send_v)

        def gather(j, rows_p, rows_q, sem_p, sem_q):
            off = j * GATHER_CHUNK
            pltpu.async_copy(
                p_hbm.at[recv_v.at[pl.ds(off, GATHER_CHUNK)]], rows_p, sem_p)
            pltpu.async_copy(
                q_hbm.at[send_v.at[pl.ds(off, GATHER_CHUNK)]], rows_q, sem_q)

        def wait_gather(j, rows_p, rows_q, sem_p, sem_q):
            off = j * GATHER_CHUNK
            pltpu.make_async_copy(
                p_hbm.at[recv_v.at[pl.ds(off, GATHER_CHUNK)]], rows_p,
                sem_p).wait()
            pltpu.make_async_copy(
                q_hbm.at[send_v.at[pl.ds(off, GATHER_CHUNK)]], rows_q,
                sem_q).wait()

        def add_rows(rows_p, rows_q):
            def row(r, carry):
                for cc in range(HIDDEN // LANES):
                    sl = pl.ds(cc * LANES, LANES)
                    plsc.addupdate(rows_p.at[r, sl], rows_q[r, sl])
                return carry
            lax.fori_loop(0, GATHER_CHUNK, row, 0, unroll=4)

        def store(j, rows_p, sem_w):
            off = base + j * GATHER_CHUNK
            pltpu.async_copy(rows_p, g_hbm.at[pl.ds(off, GATHER_CHUNK)],
                             sem_w)

        def wait_store(j, rows_p, sem_w):
            off = base + j * GATHER_CHUNK
            pltpu.make_async_copy(
                rows_p, g_hbm.at[pl.ds(off, GATHER_CHUNK)], sem_w).wait()

        gather(0, rows_pa, rows_qa, sem_pa, sem_qa)
        gather(1, rows_pb, rows_qb, sem_pb, sem_qb)

        def body(i, carry):
            j = 2 * i
            wait_gather(j, rows_pa, rows_qa, sem_pa, sem_qa)
            add_rows(rows_pa, rows_qa)
            store(j, rows_pa, sem_wa)
            wait_gather(j + 1, rows_pb, rows_qb, sem_pb, sem_qb)
            add_rows(rows_pb, rows_qb)
            store(j + 1, rows_pb, sem_wb)
            wait_store(j, rows_pa, sem_wa)
            gather(j + 2, rows_pa, rows_qa, sem_pa, sem_qa)
            wait_store(j + 1, rows_pb, sem_wb)
            gather(j + 3, rows_pb, rows_qb, sem_pb, sem_qb)
            return carry

        if full_chunks % 2:
            lax.fori_loop(0, (full_chunks - 3) // 2, body, 0, unroll=False)
            j = full_chunks - 3
            wait_gather(j, rows_pa, rows_qa, sem_pa, sem_qa)
            add_rows(rows_pa, rows_qa)
            store(j, rows_pa, sem_wa)
            wait_gather(j + 1, rows_pb, rows_qb, sem_pb, sem_qb)
            add_rows(rows_pb, rows_qb)
            store(j + 1, rows_pb, sem_wb)
            wait_store(j, rows_pa, sem_wa)
            gather(j + 2, rows_pa, rows_qa, sem_pa, sem_qa)
            wait_gather(j + 2, rows_pa, rows_qa, sem_pa, sem_qa)
            add_rows(rows_pa, rows_qa)
            store(j + 2, rows_pa, sem_wa)
            wait_store(j + 1, rows_pb, sem_wb)
            wait_store(j + 2, rows_pa, sem_wa)
        else:
            lax.fori_loop(0, (full_chunks - 2) // 2, body, 0, unroll=False)
            j = full_chunks - 2
            wait_gather(j, rows_pa, rows_qa, sem_pa, sem_qa)
            add_rows(rows_pa, rows_qa)
            store(j, rows_pa, sem_wa)
            wait_gather(j + 1, rows_pb, rows_qb, sem_pb, sem_qb)
            add_rows(rows_pb, rows_qb)
            store(j + 1, rows_pb, sem_wb)
            wait_store(j, rows_pa, sem_wa)
            wait_store(j + 1, rows_pb, sem_wb)

    return sc_body


def _sc_gather(p, q, receivers, senders, n_edges_call):
    worker_edges = n_edges_call // NUM_WORKERS
    mesh = plsc.VectorSubcoreMesh(core_axis_name="c", subcore_axis_name="s",
                                  num_cores=NUM_CORES,
                                  num_subcores=NUM_SUBCORES)
    return pl.kernel(
        _make_sc_body(worker_edges),
        out_type=jax.ShapeDtypeStruct((n_edges_call, HIDDEN), jnp.float32),
        mesh=mesh,
        scratch_types=[
            pltpu.VMEM((worker_edges,), jnp.int32),
            pltpu.VMEM((worker_edges,), jnp.int32),
            pltpu.VMEM((GATHER_CHUNK, HIDDEN), jnp.float32),
            pltpu.VMEM((GATHER_CHUNK, HIDDEN), jnp.float32),
            pltpu.VMEM((GATHER_CHUNK, HIDDEN), jnp.float32),
            pltpu.VMEM((GATHER_CHUNK, HIDDEN), jnp.float32),
            pltpu.SemaphoreType.DMA,
            pltpu.SemaphoreType.DMA,
            pltpu.SemaphoreType.DMA,
            pltpu.SemaphoreType.DMA,
            pltpu.SemaphoreType.DMA,
            pltpu.SemaphoreType.DMA,
        ],
    )(p, q, receivers, senders)


EDGE_BLOCK = 3200


def _mlp_compute(g_ref, edges_t_ref, w1e_ref, w2_ref, b2_ref, out_ref):
    t = lax.dot_general(edges_t_ref[...], w1e_ref[...],
                        dimension_numbers=((([0]), ([0])), ((), ())),
                        preferred_element_type=jnp.float32)
    h = jnp.maximum(g_ref[...] + t, 0.0)
    out_t = lax.dot_general(w2_ref[...], h,
                            dimension_numbers=((([0]), ([1])), ((), ())),
                            preferred_element_type=jnp.float32)
    out_ref[...] = out_t + b2_ref[:, 0:1]


def _mlp_first_body(g_ref, edges_t_ref, w1e_ref, w2_ref, b2_ref, out_ref):
    _mlp_compute(g_ref, edges_t_ref, w1e_ref, w2_ref, b2_ref, out_ref)


def _mlp_chain_body(g_ref, edges_t_ref, w1e_ref, w2_ref, b2_ref, prev_ref,
                    out_ref):
    _mlp_compute(g_ref, edges_t_ref, w1e_ref, w2_ref, b2_ref, out_ref)


def _edge_mlp(g, edges_t, w1e, w2, b2col, n_edges_call, blk0, prev):
    grid = (n_edges_call // EDGE_BLOCK,)
    common = dict(
        grid=grid,
        out_specs=pl.BlockSpec((D_EDGE, EDGE_BLOCK), lambda i: (0, blk0 + i)),
        out_shape=jax.ShapeDtypeStruct((D_EDGE, N_EDGES), jnp.float32),
        compiler_params=pltpu.CompilerParams(
            dimension_semantics=("arbitrary",)),
    )
    in_specs = [
        pl.BlockSpec((EDGE_BLOCK, HIDDEN), lambda i: (i, 0)),
        pl.BlockSpec((D_EDGE, EDGE_BLOCK), lambda i: (0, blk0 + i)),
        pl.BlockSpec((D_EDGE, HIDDEN), lambda i: (0, 0)),
        pl.BlockSpec((HIDDEN, D_EDGE), lambda i: (0, 0)),
        pl.BlockSpec((D_EDGE, 128), lambda i: (0, 0)),
    ]
    if prev is None:
        return pl.pallas_call(
            _mlp_first_body, in_specs=in_specs, **common,
        )(g, edges_t, w1e, w2, b2col)
    in_specs.append(pl.BlockSpec((D_EDGE, 128), lambda i: (0, 0)))
    return pl.pallas_call(
        _mlp_chain_body, in_specs=in_specs, input_output_aliases={5: 0},
        **common,
    )(g, edges_t, w1e, w2, b2col, prev)


@jax.jit
def kernel(edges, nodes, glbls, W1, b1, W2, b2, senders, receivers):
    w1e = W1[:D_EDGE]
    w1r = W1[D_EDGE:D_EDGE + D_NODE]
    w1s = W1[D_EDGE + D_NODE:D_EDGE + 2 * D_NODE]
    w1g = W1[D_EDGE + 2 * D_NODE:]
    p, q = _precompute(nodes, w1r, w1s, w1g, glbls, b1.reshape(1, HIDDEN))
    b2col = jnp.broadcast_to(b2.reshape(D_EDGE, 1), (D_EDGE, 128))
    edges_t = edges.T

    out_t = None
    lo = 0
    for size in SLICE_SIZES:
        rc = lax.slice_in_dim(receivers, lo, lo + size)
        sc = lax.slice_in_dim(senders, lo, lo + size)
        g = _sc_gather(p, q, rc, sc, size)
        out_t = _edge_mlp(g, edges_t, w1e, W2, b2col, size,
                          lo // EDGE_BLOCK, out_t)
        lo += size
    return out_t.T


# speedup vs baseline: 2.3950x; 1.1648x over previous
import jax
import jax.numpy as jnp
from jax import lax
from jax.experimental import pallas as pl
from jax.experimental.pallas import tpu as pltpu
from jax.experimental.pallas import tpu_sc as plsc

N_NODES = 10000
N_EDGES = 320000
D_NODE = 128
D_EDGE = 16
D_GLOBAL = 64
HIDDEN = 128
LANES = 16

NUM_CORES = 2
NUM_SUBCORES = 16
NUM_WORKERS = NUM_CORES * NUM_SUBCORES

SLICE_SIZES = (160000, 96000, 64000)
GATHER_CHUNK = 40


def _precompute_body(nodes_ref, w1r_ref, w1s_ref, w1g_ref, glbls_ref, b1_ref,
                     p_ref, q_ref):
    c = jnp.dot(glbls_ref[...], w1g_ref[...],
                preferred_element_type=jnp.float32) + b1_ref[...]
    nodes = nodes_ref[...]
    p_ref[...] = jnp.dot(nodes, w1r_ref[...],
                         preferred_element_type=jnp.float32) + c
    q_ref[...] = jnp.dot(nodes, w1s_ref[...],
                         preferred_element_type=jnp.float32)


def _precompute(nodes, w1r, w1s, w1g, glbls, b1):
    return pl.pallas_call(
        _precompute_body,
        out_shape=(
            jax.ShapeDtypeStruct((N_NODES, HIDDEN), jnp.float32),
            jax.ShapeDtypeStruct((N_NODES, HIDDEN), jnp.float32),
        ),
    )(nodes, w1r, w1s, w1g, glbls, b1)


def _make_sc_body(worker_edges):
    count = worker_edges // GATHER_CHUNK

    def sc_body(p_hbm, q_hbm, recv_hbm, send_hbm, g_hbm, recv_v, send_v,
                rpa, rqa, rpb, rqb, rpc, rqc, rpd, rqd,
                spa, sqa, swa, spb, sqb, swb, spc, sqc, swc, spd, sqd, swd):
        wid = lax.axis_index("s") * NUM_CORES + lax.axis_index("c")
        base = wid * worker_edges

        slot_a = (rpa, rqa, spa, sqa, swa)
        slot_b = (rpb, rqb, spb, sqb, swb)
        slot_c = (rpc, rqc, spc, sqc, swc)
        slot_d = (rpd, rqd, spd, sqd, swd)

        pltpu.sync_copy(recv_hbm.at[pl.ds(base, worker_edges)], recv_v)
        pltpu.sync_copy(send_hbm.at[pl.ds(base, worker_edges)], send_v)

        def gather(j, slot):
            rows_p, rows_q, sem_p, sem_q, _ = slot
            off = j * GATHER_CHUNK
            pltpu.async_copy(
                p_hbm.at[recv_v.at[pl.ds(off, GATHER_CHUNK)]], rows_p, sem_p)
            pltpu.async_copy(
                q_hbm.at[send_v.at[pl.ds(off, GATHER_CHUNK)]], rows_q, sem_q)

        def wait_gather(j, slot):
            rows_p, rows_q, sem_p, sem_q, _ = slot
            off = j * GATHER_CHUNK
            pltpu.make_async_copy(
                p_hbm.at[recv_v.at[pl.ds(off, GATHER_CHUNK)]], rows_p,
                sem_p).wait()
            pltpu.make_async_copy(
                q_hbm.at[send_v.at[pl.ds(off, GATHER_CHUNK)]], rows_q,
                sem_q).wait()

        def add_rows(slot):
            rows_p, rows_q = slot[0], slot[1]

            def row(r, carry):
                for cc in range(HIDDEN // LANES):
                    sl = pl.ds(cc * LANES, LANES)
                    plsc.addupdate(rows_p.at[r, sl], rows_q[r, sl])
                return carry
            lax.fori_loop(0, GATHER_CHUNK, row, 0, unroll=4)

        def store(j, slot):
            off = base + j * GATHER_CHUNK
            pltpu.async_copy(slot[0], g_hbm.at[pl.ds(off, GATHER_CHUNK)],
                             slot[4])

        def wait_store(j, slot):
            off = base + j * GATHER_CHUNK
            pltpu.make_async_copy(
                slot[0], g_hbm.at[pl.ds(off, GATHER_CHUNK)], slot[4]).wait()

        gather(0, slot_a)
        gather(1, slot_b)
        gather(2, slot_c)

        def body(i, carry):
            j = 4 * i
            wait_gather(j, slot_a)
            add_rows(slot_a)
            store(j, slot_a)
            gather(j + 3, slot_d)
            wait_gather(j + 1, slot_b)
            add_rows(slot_b)
            store(j + 1, slot_b)
            wait_store(j, slot_a)
            gather(j + 4, slot_a)
            wait_gather(j + 2, slot_c)
            add_rows(slot_c)
            store(j + 2, slot_c)
            wait_store(j + 1, slot_b)
            gather(j + 5, slot_b)
            wait_gather(j + 3, slot_d)
            add_rows(slot_d)
            store(j + 3, slot_d)
            wait_store(j + 2, slot_c)
            gather(j + 6, slot_c)
            wait_store(j + 3, slot_d)
            return carry

        nbody = (count - 3) // 4
        lax.fori_loop(0, nbody, body, 0, unroll=False)

        inflight = [(slot_a, 4 * nbody), (slot_b, 4 * nbody + 1),
                    (slot_c, 4 * nbody + 2)]
        togather = list(range(4 * nbody + 3, count))
        free = [slot_d]
        outstanding = {}
        while inflight:
            slot, j = inflight.pop(0)
            wait_gather(j, slot)
            add_rows(slot)
            store(j, slot)
            outstanding[id(slot)] = (j, slot)
            if togather:
                jj = togather.pop(0)
                fs = free.pop(0)
                if id(fs) in outstanding:
                    pj, _ = outstanding.pop(id(fs))
                    wait_store(pj, fs)
                gather(jj, fs)
                inflight.append((fs, jj))
            free.append(slot)
        for j, slot in outstanding.values():
            wait_store(j, slot)

    return sc_body


def _sc_gather(p, q, receivers, senders, n_edges_call):
    worker_edges = n_edges_call // NUM_WORKERS
    mesh = plsc.VectorSubcoreMesh(core_axis_name="c", subcore_axis_name="s",
                                  num_cores=NUM_CORES,
                                  num_subcores=NUM_SUBCORES)
    row_buf = pltpu.VMEM((GATHER_CHUNK, HIDDEN), jnp.float32)
    return pl.kernel(
        _make_sc_body(worker_edges),
        out_type=jax.ShapeDtypeStruct((n_edges_call, HIDDEN), jnp.float32),
        mesh=mesh,
        scratch_types=(
            [pltpu.VMEM((worker_edges,), jnp.int32)] * 2
            + [row_buf] * 8
            + [pltpu.SemaphoreType.DMA] * 12
        ),
    )(p, q, receivers, senders)


EDGE_BLOCK = 3200


def _mlp_compute(g_ref, edges_t_ref, w1e_ref, w2_ref, b2_ref, out_ref):
    t = lax.dot_general(edges_t_ref[...], w1e_ref[...],
                        dimension_numbers=((([0]), ([0])), ((), ())),
                        preferred_element_type=jnp.float32)
    h = jnp.maximum(g_ref[...] + t, 0.0)
    out_t = lax.dot_general(w2_ref[...], h,
                            dimension_numbers=((([0]), ([1])), ((), ())),
                            preferred_element_type=jnp.float32)
    out_ref[...] = out_t + b2_ref[:, 0:1]


def _mlp_first_body(g_ref, edges_t_ref, w1e_ref, w2_ref, b2_ref, out_ref):
    _mlp_compute(g_ref, edges_t_ref, w1e_ref, w2_ref, b2_ref, out_ref)


def _mlp_chain_body(g_ref, edges_t_ref, w1e_ref, w2_ref, b2_ref, prev_ref,
                    out_ref):
    _mlp_compute(g_ref, edges_t_ref, w1e_ref, w2_ref, b2_ref, out_ref)


def _edge_mlp(g, edges_t, w1e, w2, b2col, n_edges_call, blk0, prev):
    grid = (n_edges_call // EDGE_BLOCK,)
    common = dict(
        grid=grid,
        out_specs=pl.BlockSpec((D_EDGE, EDGE_BLOCK), lambda i: (0, blk0 + i)),
        out_shape=jax.ShapeDtypeStruct((D_EDGE, N_EDGES), jnp.float32),
        compiler_params=pltpu.CompilerParams(
            dimension_semantics=("arbitrary",)),
    )
    in_specs = [
        pl.BlockSpec((EDGE_BLOCK, HIDDEN), lambda i: (i, 0)),
        pl.BlockSpec((D_EDGE, EDGE_BLOCK), lambda i: (0, blk0 + i)),
        pl.BlockSpec((D_EDGE, HIDDEN), lambda i: (0, 0)),
        pl.BlockSpec((HIDDEN, D_EDGE), lambda i: (0, 0)),
        pl.BlockSpec((D_EDGE, 128), lambda i: (0, 0)),
    ]
    if prev is None:
        return pl.pallas_call(
            _mlp_first_body, in_specs=in_specs, **common,
        )(g, edges_t, w1e, w2, b2col)
    in_specs.append(pl.BlockSpec((D_EDGE, 128), lambda i: (0, 0)))
    return pl.pallas_call(
        _mlp_chain_body, in_specs=in_specs, input_output_aliases={5: 0},
        **common,
    )(g, edges_t, w1e, w2, b2col, prev)


@jax.jit
def kernel(edges, nodes, glbls, W1, b1, W2, b2, senders, receivers):
    w1e = W1[:D_EDGE]
    w1r = W1[D_EDGE:D_EDGE + D_NODE]
    w1s = W1[D_EDGE + D_NODE:D_EDGE + 2 * D_NODE]
    w1g = W1[D_EDGE + 2 * D_NODE:]
    p, q = _precompute(nodes, w1r, w1s, w1g, glbls, b1.reshape(1, HIDDEN))
    b2col = jnp.broadcast_to(b2.reshape(D_EDGE, 1), (D_EDGE, 128))
    edges_t = edges.T

    out_t = None
    lo = 0
    for size in SLICE_SIZES:
        rc = lax.slice_in_dim(receivers, lo, lo + size)
        sc = lax.slice_in_dim(senders, lo, lo + size)
        g = _sc_gather(p, q, rc, sc, size)
        out_t = _edge_mlp(g, edges_t, w1e, W2, b2col, size,
                          lo // EDGE_BLOCK, out_t)
        lo += size
    return out_t.T


# speedup vs baseline: 2.3978x; 1.0012x over previous
import jax
import jax.numpy as jnp
from jax import lax
from jax.experimental import pallas as pl
from jax.experimental.pallas import tpu as pltpu
from jax.experimental.pallas import tpu_sc as plsc

N_NODES = 10000
N_EDGES = 320000
D_NODE = 128
D_EDGE = 16
D_GLOBAL = 64
HIDDEN = 128
LANES = 16

NUM_CORES = 2
NUM_SUBCORES = 16
NUM_WORKERS = NUM_CORES * NUM_SUBCORES

SLICE_SIZES = (128000, 128000, 64000)
GATHER_CHUNK = 40


def _precompute_body(nodes_ref, w1r_ref, w1s_ref, w1g_ref, glbls_ref, b1_ref,
                     p_ref, q_ref):
    c = jnp.dot(glbls_ref[...], w1g_ref[...],
                preferred_element_type=jnp.float32) + b1_ref[...]
    nodes = nodes_ref[...]
    p_ref[...] = jnp.dot(nodes, w1r_ref[...],
                         preferred_element_type=jnp.float32) + c
    q_ref[...] = jnp.dot(nodes, w1s_ref[...],
                         preferred_element_type=jnp.float32)


def _precompute(nodes, w1r, w1s, w1g, glbls, b1):
    return pl.pallas_call(
        _precompute_body,
        out_shape=(
            jax.ShapeDtypeStruct((N_NODES, HIDDEN), jnp.float32),
            jax.ShapeDtypeStruct((N_NODES, HIDDEN), jnp.float32),
        ),
    )(nodes, w1r, w1s, w1g, glbls, b1)


def _make_sc_body(worker_edges):
    count = worker_edges // GATHER_CHUNK

    def sc_body(p_hbm, q_hbm, recv_hbm, send_hbm, g_hbm, recv_v, send_v,
                rpa, rqa, rpb, rqb, rpc, rqc, rpd, rqd,
                spa, sqa, swa, spb, sqb, swb, spc, sqc, swc, spd, sqd, swd):
        wid = lax.axis_index("s") * NUM_CORES + lax.axis_index("c")
        base = wid * worker_edges

        slot_a = (rpa, rqa, spa, sqa, swa)
        slot_b = (rpb, rqb, spb, sqb, swb)
        slot_c = (rpc, rqc, spc, sqc, swc)
        slot_d = (rpd, rqd, spd, sqd, swd)

        pltpu.sync_copy(recv_hbm.at[pl.ds(base, worker_edges)], recv_v)
        pltpu.sync_copy(send_hbm.at[pl.ds(base, worker_edges)], send_v)

        def gather(j, slot):
            rows_p, rows_q, sem_p, sem_q, _ = slot
            off = j * GATHER_CHUNK
            pltpu.async_copy(
                p_hbm.at[recv_v.at[pl.ds(off, GATHER_CHUNK)]], rows_p, sem_p)
            pltpu.async_copy(
                q_hbm.at[send_v.at[pl.ds(off, GATHER_CHUNK)]], rows_q, sem_q)

        def wait_gather(j, slot):
            rows_p, rows_q, sem_p, sem_q, _ = slot
            off = j * GATHER_CHUNK
            pltpu.make_async_copy(
                p_hbm.at[recv_v.at[pl.ds(off, GATHER_CHUNK)]], rows_p,
                sem_p).wait()
            pltpu.make_async_copy(
                q_hbm.at[send_v.at[pl.ds(off, GATHER_CHUNK)]], rows_q,
                sem_q).wait()

        def add_rows(slot):
            rows_p, rows_q = slot[0], slot[1]

            def row(r, carry):
                for cc in range(HIDDEN // LANES):
                    sl = pl.ds(cc * LANES, LANES)
                    plsc.addupdate(rows_p.at[r, sl], rows_q[r, sl])
                return carry
            lax.fori_loop(0, GATHER_CHUNK, row, 0, unroll=4)

        def store(j, slot):
            off = base + j * GATHER_CHUNK
            pltpu.async_copy(slot[0], g_hbm.at[pl.ds(off, GATHER_CHUNK)],
                             slot[4])

        def wait_store(j, slot):
            off = base + j * GATHER_CHUNK
            pltpu.make_async_copy(
                slot[0], g_hbm.at[pl.ds(off, GATHER_CHUNK)], slot[4]).wait()

        gather(0, slot_a)
        gather(1, slot_b)
        gather(2, slot_c)

        def body(i, carry):
            j = 4 * i
            wait_gather(j, slot_a)
            add_rows(slot_a)
            store(j, slot_a)
            gather(j + 3, slot_d)
            wait_gather(j + 1, slot_b)
            add_rows(slot_b)
            store(j + 1, slot_b)
            wait_store(j, slot_a)
            gather(j + 4, slot_a)
            wait_gather(j + 2, slot_c)
            add_rows(slot_c)
            store(j + 2, slot_c)
            wait_store(j + 1, slot_b)
            gather(j + 5, slot_b)
            wait_gather(j + 3, slot_d)
            add_rows(slot_d)
            store(j + 3, slot_d)
            wait_store(j + 2, slot_c)
            gather(j + 6, slot_c)
            wait_store(j + 3, slot_d)
            return carry

        nbody = (count - 3) // 4
        lax.fori_loop(0, nbody, body, 0, unroll=False)

        inflight = [(slot_a, 4 * nbody), (slot_b, 4 * nbody + 1),
                    (slot_c, 4 * nbody + 2)]
        togather = list(range(4 * nbody + 3, count))
        free = [slot_d]
        outstanding = {}
        while inflight:
            slot, j = inflight.pop(0)
            wait_gather(j, slot)
            add_rows(slot)
            store(j, slot)
            outstanding[id(slot)] = (j, slot)
            if togather:
                jj = togather.pop(0)
                fs = free.pop(0)
                if id(fs) in outstanding:
                    pj, _ = outstanding.pop(id(fs))
                    wait_store(pj, fs)
                gather(jj, fs)
                inflight.append((fs, jj))
            free.append(slot)
        for j, slot in outstanding.values():
            wait_store(j, slot)

    return sc_body


def _sc_gather(p, q, receivers, senders, n_edges_call):
    worker_edges = n_edges_call // NUM_WORKERS
    mesh = plsc.VectorSubcoreMesh(core_axis_name="c", subcore_axis_name="s",
                                  num_cores=NUM_CORES,
                                  num_subcores=NUM_SUBCORES)
    row_buf = pltpu.VMEM((GATHER_CHUNK, HIDDEN), jnp.float32)
    return pl.kernel(
        _make_sc_body(worker_edges),
        out_type=jax.ShapeDtypeStruct((n_edges_call, HIDDEN), jnp.float32),
        mesh=mesh,
        scratch_types=(
            [pltpu.VMEM((worker_edges,), jnp.int32)] * 2
            + [row_buf] * 8
            + [pltpu.SemaphoreType.DMA] * 12
        ),
    )(p, q, receivers, senders)


EDGE_BLOCK = 3200


def _mlp_compute(g_ref, edges_t_ref, w1e_ref, w2_ref, b2_ref, out_ref):
    t = lax.dot_general(edges_t_ref[...], w1e_ref[...],
                        dimension_numbers=((([0]), ([0])), ((), ())),
                        preferred_element_type=jnp.float32)
    h = jnp.maximum(g_ref[...] + t, 0.0)
    out_t = lax.dot_general(w2_ref[...], h,
                            dimension_numbers=((([0]), ([1])), ((), ())),
                            preferred_element_type=jnp.float32)
    out_ref[...] = out_t + b2_ref[:, 0:1]


def _mlp_first_body(g_ref, edges_t_ref, w1e_ref, w2_ref, b2_ref, out_ref):
    _mlp_compute(g_ref, edges_t_ref, w1e_ref, w2_ref, b2_ref, out_ref)


def _mlp_chain_body(g_ref, edges_t_ref, w1e_ref, w2_ref, b2_ref, prev_ref,
                    out_ref):
    _mlp_compute(g_ref, edges_t_ref, w1e_ref, w2_ref, b2_ref, out_ref)


def _edge_mlp(g, edges_t, w1e, w2, b2col, n_edges_call, blk0, prev):
    grid = (n_edges_call // EDGE_BLOCK,)
    common = dict(
        grid=grid,
        out_specs=pl.BlockSpec((D_EDGE, EDGE_BLOCK), lambda i: (0, blk0 + i)),
        out_shape=jax.ShapeDtypeStruct((D_EDGE, N_EDGES), jnp.float32),
        compiler_params=pltpu.CompilerParams(
            dimension_semantics=("arbitrary",)),
    )
    in_specs = [
        pl.BlockSpec((EDGE_BLOCK, HIDDEN), lambda i: (i, 0)),
        pl.BlockSpec((D_EDGE, EDGE_BLOCK), lambda i: (0, blk0 + i)),
        pl.BlockSpec((D_EDGE, HIDDEN), lambda i: (0, 0)),
        pl.BlockSpec((HIDDEN, D_EDGE), lambda i: (0, 0)),
        pl.BlockSpec((D_EDGE, 128), lambda i: (0, 0)),
    ]
    if prev is None:
        return pl.pallas_call(
            _mlp_first_body, in_specs=in_specs, **common,
        )(g, edges_t, w1e, w2, b2col)
    in_specs.append(pl.BlockSpec((D_EDGE, 128), lambda i: (0, 0)))
    return pl.pallas_call(
        _mlp_chain_body, in_specs=in_specs, input_output_aliases={5: 0},
        **common,
    )(g, edges_t, w1e, w2, b2col, prev)


@jax.jit
def kernel(edges, nodes, glbls, W1, b1, W2, b2, senders, receivers):
    w1e = W1[:D_EDGE]
    w1r = W1[D_EDGE:D_EDGE + D_NODE]
    w1s = W1[D_EDGE + D_NODE:D_EDGE + 2 * D_NODE]
    w1g = W1[D_EDGE + 2 * D_NODE:]
    p, q = _precompute(nodes, w1r, w1s, w1g, glbls, b1.reshape(1, HIDDEN))
    b2col = jnp.broadcast_to(b2.reshape(D_EDGE, 1), (D_EDGE, 128))
    edges_t = edges.T

    out_t = None
    lo = 0
    for size in SLICE_SIZES:
        rc = lax.slice_in_dim(receivers, lo, lo + size)
        sc = lax.slice_in_dim(senders, lo, lo + size)
        g = _sc_gather(p, q, rc, sc, size)
        out_t = _edge_mlp(g, edges_t, w1e, W2, b2col, size,
                          lo // EDGE_BLOCK, out_t)
        lo += size
    return out_t.T
